# Initial kernel scaffold; baseline (speedup 1.0000x reference)
import jax
import jax.numpy as jnp
from jax.experimental import pallas as pl


def kernel(x):
    raise NotImplementedError("write your pallas kernel here")


# speedup vs baseline: 28.2374x; 28.2374x over previous
import jax
import jax.numpy as jnp
from jax.experimental import pallas as pl
from jax.experimental.pallas import tpu as pltpu

_K = 512


def _topk_mask_body(x_ref, o_ref):
    x = x_ref[...]
    u = jax.lax.bitcast_convert_type(x, jnp.uint32)
    s = u >> jnp.uint32(31)
    key = u ^ (s * jnp.uint32(0x7FFFFFFF) + jnp.uint32(0x80000000))

    prefix = jnp.zeros((x.shape[0], 1), jnp.uint32)
    for b in range(31, -1, -1):
        cand = prefix | jnp.uint32(1 << b)
        cnt = jnp.sum((key >= cand).astype(jnp.int32), axis=1, keepdims=True)
        prefix = jnp.where(cnt >= _K, cand, prefix)

    o_ref[...] = jnp.where(key >= prefix, x, jnp.float32(0.0))


def kernel(x):
    return pl.pallas_call(
        _topk_mask_body,
        out_shape=jax.ShapeDtypeStruct(x.shape, x.dtype),
    )(x)
